# baseline (speedup 1.0000x reference)
import functools

import jax
import jax.numpy as jnp
from jax import lax
from jax.experimental import pallas as pl
from jax.experimental.pallas import tpu as pltpu
from jax.experimental.pallas import tpu_sc as plsc

POI = 1000000
D = 64
B = 4096
H = 200
TOT = B * H
CH = 128
NBUF = 4

NC = 2
NS = 16
NW = NC * NS
NCH = TOT // (NW * CH)
NG = NCH // NBUF


@functools.lru_cache(maxsize=1)
def _build_gather():
    mesh = plsc.VectorSubcoreMesh(core_axis_name="c", subcore_axis_name="s")

    @functools.partial(
        pl.kernel,
        mesh=mesh,
        out_type=jax.ShapeDtypeStruct((TOT, 2 * D), jnp.float32),
        compiler_params=pltpu.CompilerParams(
            use_tc_tiling_on_sc=True, needs_layout_passes=False),
        scratch_types=(
            [pltpu.VMEM((NCH, CH), jnp.int32)]
            + [pltpu.VMEM((CH, 2 * D), jnp.float32) for _ in range(NBUF)]
            + [pltpu.SemaphoreType.DMA for _ in range(2 * NBUF)]
        ),
    )
    def gather_kernel(table_hbm, idx_hbm, out_hbm, idx_v, *rest):
        rows = rest[:NBUF]
        gsem = rest[NBUF:2 * NBUF]
        wsem = rest[2 * NBUF:]

        wid = lax.axis_index("s") * NC + lax.axis_index("c")
        pltpu.sync_copy(idx_hbm.at[pl.ds(wid * NCH, NCH)], idx_v)
        out_base = wid * NCH * CH

        for b in range(NBUF):
            pltpu.async_copy(table_hbm.at[idx_v.at[b]], rows[b], gsem[b])

        def group(g, carry):
            for b in range(NBUF):
                j = g * NBUF + b
                pltpu.make_async_copy(
                    table_hbm.at[idx_v.at[b]], rows[b], gsem[b]).wait()
                row0 = pl.multiple_of(out_base + j * CH, CH)
                pltpu.async_copy(
                    rows[b], out_hbm.at[pl.ds(row0, CH)], wsem[b])

                @pl.when(g < NG - 1)
                def _():
                    pltpu.make_async_copy(
                        rows[b], out_hbm.at[pl.ds(0, CH)], wsem[b]).wait()
                    pltpu.async_copy(
                        table_hbm.at[idx_v.at[j + NBUF]], rows[b], gsem[b])
            return carry

        lax.fori_loop(0, NG, group, 0)

        for b in range(NBUF):
            pltpu.make_async_copy(
                rows[b], out_hbm.at[pl.ds(0, CH)], wsem[b]).wait()

    return gather_kernel


BB = 128
HB = 8


def _select_transpose_block(x_ref, p_ref, o_ref):
    for hh in range(HB):
        pair = x_ref[:, hh, :]
        lo = pair[:, :D].T
        hi = pair[:, D:].T
        par = p_ref[hh, :]
        o_ref[hh] = jnp.where(par[None, :] == 1, hi, lo)


@functools.lru_cache(maxsize=1)
def _build_select_transpose():
    return pl.pallas_call(
        _select_transpose_block,
        grid=(B // BB, H // HB),
        in_specs=[
            pl.BlockSpec((BB, HB, 2 * D), lambda bi, hi: (bi, hi, 0)),
            pl.BlockSpec((HB, BB), lambda bi, hi: (hi, bi)),
        ],
        out_specs=pl.BlockSpec((HB, D, BB), lambda bi, hi: (hi, 0, bi)),
        out_shape=jax.ShapeDtypeStruct((H, D, B), jnp.float32),
    )


def kernel(traj, table):
    tbl2 = table.reshape(POI // 2, 2 * D)
    ti = traj.astype(jnp.int32)
    pair_idx = (ti >> 1).reshape(TOT // CH, CH)
    parityT = (ti & 1).T
    flatp = _build_gather()(tbl2, pair_idx)
    flat3 = flatp.reshape(B, H, 2 * D)
    out3 = _build_select_transpose()(flat3, parityT)
    return out3.transpose(2, 0, 1)


# speedup vs baseline: 1.3929x; 1.3929x over previous
import functools

import jax
import jax.numpy as jnp
from jax import lax
from jax.experimental import pallas as pl
from jax.experimental.pallas import tpu as pltpu
from jax.experimental.pallas import tpu_sc as plsc

POI = 1000000
D = 64
B = 4096
H = 200
TOT = B * H
CH = 128
NBUF = 4

NC = 2
NS = 16
NW = NC * NS
NCH = TOT // (NW * CH)
NG = NCH // NBUF


@functools.lru_cache(maxsize=1)
def _build_gather():
    mesh = plsc.VectorSubcoreMesh(core_axis_name="c", subcore_axis_name="s")

    @functools.partial(
        pl.kernel,
        mesh=mesh,
        out_type=jax.ShapeDtypeStruct((TOT, D), jnp.float32),
        compiler_params=pltpu.CompilerParams(use_tc_tiling_on_sc=False),
        scratch_types=(
            [pltpu.VMEM((NCH, CH), jnp.int32)]
            + [pltpu.VMEM((CH, D), jnp.float32) for _ in range(NBUF)]
            + [pltpu.SemaphoreType.DMA for _ in range(2 * NBUF)]
        ),
    )
    def gather_kernel(table_hbm, idx_hbm, out_hbm, idx_v, *rest):
        rows = rest[:NBUF]
        gsem = rest[NBUF:2 * NBUF]
        wsem = rest[2 * NBUF:]

        wid = lax.axis_index("s") * NC + lax.axis_index("c")
        pltpu.sync_copy(idx_hbm.at[pl.ds(wid * NCH, NCH)], idx_v)
        out_base = wid * NCH * CH

        for b in range(NBUF):
            pltpu.async_copy(table_hbm.at[idx_v.at[b]], rows[b], gsem[b])

        def group(g, carry):
            for b in range(NBUF):
                j = g * NBUF + b
                pltpu.make_async_copy(
                    table_hbm.at[idx_v.at[b]], rows[b], gsem[b]).wait()
                row0 = pl.multiple_of(out_base + j * CH, CH)
                pltpu.async_copy(
                    rows[b], out_hbm.at[pl.ds(row0, CH)], wsem[b])

                @pl.when(g < NG - 1)
                def _():
                    pltpu.make_async_copy(
                        rows[b], out_hbm.at[pl.ds(0, CH)], wsem[b]).wait()
                    pltpu.async_copy(
                        table_hbm.at[idx_v.at[j + NBUF]], rows[b], gsem[b])
            return carry

        lax.fori_loop(0, NG, group, 0)

        for b in range(NBUF):
            pltpu.make_async_copy(
                rows[b], out_hbm.at[pl.ds(0, CH)], wsem[b]).wait()

    return gather_kernel


BB = 128


def _transpose_block(x_ref, o_ref):
    for hh in range(H // 2):
        x = x_ref[:, hh, :]
        o_ref[2 * hh] = x[:, :D].T
        o_ref[2 * hh + 1] = x[:, D:].T


@functools.lru_cache(maxsize=1)
def _build_transpose():
    return pl.pallas_call(
        _transpose_block,
        grid=(B // BB,),
        in_specs=[pl.BlockSpec((BB, H // 2, 2 * D), lambda bi: (bi, 0, 0))],
        out_specs=pl.BlockSpec((H, D, BB), lambda bi: (0, 0, bi)),
        out_shape=jax.ShapeDtypeStruct((H, D, B), jnp.float32),
    )


def kernel(traj, table):
    flat_idx = traj.reshape(TOT // CH, CH).astype(jnp.int32)
    flat = _build_gather()(table, flat_idx)
    flat3 = flat.reshape(B, H // 2, 2 * D)
    out3 = _build_transpose()(flat3)
    return out3.transpose(2, 0, 1)
